# Initial kernel scaffold; baseline (speedup 1.0000x reference)
import functools

import jax
import jax.numpy as jnp
from jax import lax
from jax.experimental import pallas as pl
from jax.experimental.pallas import tpu as pltpu
from jax.experimental.pallas import tpu_sc as plsc

F32 = jnp.float32

N = 10000
E = 320000
E_IN = 16
D_IN = 128
LAT = 16
D_OUT = 128

CH = 128
NCHUNK = E // CH
KN = 1000
KE = 2000


def _dot(a, b):
    return lax.dot_general(a, b, (((1,), (0,)), ((), ())),
                           preferred_element_type=F32)


def _prep_body(nf_ref, ws_ref, wd_ref, b1_ref, a_ref, b_ref):
    nf = nf_ref[...]
    a_ref[...] = _dot(nf, ws_ref[...])
    b_ref[...] = _dot(nf, wd_ref[...]) + b1_ref[...]


def _emlp_body(ef_ref, ga_ref, gb_ref, w1_ref, w2_ref, b2_ref, w3_ref,
               b3_ref, out_ref):
    h = jnp.maximum(_dot(ef_ref[...], w1_ref[...]) + ga_ref[...] + gb_ref[...], 0.0)
    h = jnp.maximum(_dot(h, w2_ref[...]) + b2_ref[...], 0.0)
    z = _dot(h, w3_ref[...]) + b3_ref[...]
    out_ref[...] = 1.0 / (1.0 + jnp.exp(-z))


def _nmlp_body(nf_ref, ps_ref, pd_ref, w1t_ref, w1b_ref, b1_ref, w2_ref,
               b2_ref, w3_ref, b3_ref, out_ref):
    i = pl.program_id(0)
    s = ps_ref[0] + ps_ref[1]
    d = pd_ref[0, pl.ds(i * KN, KN)] + pd_ref[1, pl.ds(i * KN, KN)]
    h_n = s * (1.0 / jnp.maximum(d, 1.0))[:, None]
    h = jnp.maximum(_dot(nf_ref[...], w1t_ref[...]) + _dot(h_n, w1b_ref[...])
                    + b1_ref[...], 0.0)
    h = jnp.maximum(_dot(h, w2_ref[...]) + b2_ref[...], 0.0)
    z = _dot(h, w3_ref[...]) + b3_ref[...]
    out_ref[...] = 1.0 / (1.0 + jnp.exp(-z))


def _gather_body(nc, nw, nfull, rem,
                 src_hbm, dst_hbm, a_hbm, b_hbm, ga_hbm, gb_hbm,
                 sv, dv, ba, bb, sem_a, sem_b):
    wid = lax.axis_index("s") * nc + lax.axis_index("c")
    n_my = nfull + jnp.where(wid < rem, 1, 0)

    def chunk(j, carry):
        off = CH * (wid + nw * j)
        pltpu.sync_copy(src_hbm.at[pl.ds(off, CH)], sv)
        pltpu.sync_copy(dst_hbm.at[pl.ds(off, CH)], dv)
        ca = pltpu.async_copy(a_hbm.at[sv], ba, sem_a)
        cb = pltpu.async_copy(b_hbm.at[dv], bb, sem_b)
        ca.wait()
        cb.wait()
        pltpu.sync_copy(ba, ga_hbm.at[pl.ds(off, CH)])
        pltpu.sync_copy(bb, gb_hbm.at[pl.ds(off, CH)])
        return carry

    lax.fori_loop(0, n_my, chunk, 0)


def _scatter_body(nc, nw, nfull, rem,
                  dst2d_hbm, eo_hbm, z2d_hbm, z1d_hbm, ones_hbm,
                  psum_hbm, pdeg_hbm,
                  sums_sp, deg_sp, idxv, rows, onesv):
    cid = lax.axis_index("c")
    sid = lax.axis_index("s")
    wid = sid * nc + cid
    n_my = nfull + jnp.where(wid < rem, 1, 0)

    for k in range(5):
        pltpu.sync_copy(z2d_hbm, sums_sp.at[pl.ds(625 * sid + 125 * k, 125)])

    @pl.when(sid == 0)
    def _():
        for k in range(5):
            pltpu.sync_copy(z1d_hbm, deg_sp.at[pl.ds(2000 * k, 2000)])

    pltpu.sync_copy(ones_hbm, onesv)
    plsc.subcore_barrier()

    def chunk(j, carry):
        c = wid + nw * j
        pltpu.sync_copy(dst2d_hbm.at[c], idxv.at[0])
        pltpu.sync_copy(eo_hbm.at[pl.ds(CH * c, CH)], rows)
        pltpu.sync_copy(rows, sums_sp.at[idxv.at[0]], add=True)
        pltpu.sync_copy(onesv, deg_sp.at[idxv.at[0]], add=True)
        return carry

    lax.fori_loop(0, n_my, chunk, 0)
    plsc.subcore_barrier()

    pltpu.sync_copy(sums_sp.at[pl.ds(625 * sid, 625)],
                    psum_hbm.at[cid, pl.ds(625 * sid, 625)])

    @pl.when(sid < 10)
    def _():
        pltpu.sync_copy(deg_sp.at[pl.ds(1000 * sid, 1000)],
                        pdeg_hbm.at[cid, pl.ds(1000 * sid, 1000)])


def kernel(n_feats, e_feats, edge_index, W1e, b1e, W2e, b2e, W3e, b3e,
           W1n, b1n, W2n, b2n, W3n, b3n):
    src = edge_index[0]
    dst = edge_index[1]

    a_tab, b_tab = pl.pallas_call(
        _prep_body,
        grid=(1,),
        in_specs=[
            pl.BlockSpec((N, D_IN), lambda i: (0, 0)),
            pl.BlockSpec((D_IN, LAT), lambda i: (0, 0)),
            pl.BlockSpec((D_IN, LAT), lambda i: (0, 0)),
            pl.BlockSpec((1, LAT), lambda i: (0, 0)),
        ],
        out_specs=[
            pl.BlockSpec((N, LAT), lambda i: (0, 0)),
            pl.BlockSpec((N, LAT), lambda i: (0, 0)),
        ],
        out_shape=[
            jax.ShapeDtypeStruct((N, LAT), F32),
            jax.ShapeDtypeStruct((N, LAT), F32),
        ],
    )(n_feats, W1e[E_IN:E_IN + D_IN], W1e[E_IN + D_IN:], b1e.reshape(1, LAT))

    info = plsc.get_sparse_core_info()
    nc, ns = info.num_cores, info.num_subcores
    nw = nc * ns
    nfull, rem = NCHUNK // nw, NCHUNK % nw
    mesh = plsc.VectorSubcoreMesh(core_axis_name="c", subcore_axis_name="s")

    ga, gb = pl.kernel(
        functools.partial(_gather_body, nc, nw, nfull, rem),
        out_type=[
            jax.ShapeDtypeStruct((E, LAT), F32),
            jax.ShapeDtypeStruct((E, LAT), F32),
        ],
        mesh=mesh,
        scratch_types=[
            pltpu.VMEM((CH,), jnp.int32),
            pltpu.VMEM((CH,), jnp.int32),
            pltpu.VMEM((CH, LAT), F32),
            pltpu.VMEM((CH, LAT), F32),
            pltpu.SemaphoreType.DMA,
            pltpu.SemaphoreType.DMA,
        ],
    )(src, dst, a_tab, b_tab)

    e_out = pl.pallas_call(
        _emlp_body,
        grid=(E // KE,),
        in_specs=[
            pl.BlockSpec((KE, E_IN), lambda i: (i, 0)),
            pl.BlockSpec((KE, LAT), lambda i: (i, 0)),
            pl.BlockSpec((KE, LAT), lambda i: (i, 0)),
            pl.BlockSpec((E_IN, LAT), lambda i: (0, 0)),
            pl.BlockSpec((LAT, LAT), lambda i: (0, 0)),
            pl.BlockSpec((1, LAT), lambda i: (0, 0)),
            pl.BlockSpec((LAT, D_OUT), lambda i: (0, 0)),
            pl.BlockSpec((1, D_OUT), lambda i: (0, 0)),
        ],
        out_specs=pl.BlockSpec((KE, D_OUT), lambda i: (i, 0)),
        out_shape=jax.ShapeDtypeStruct((E, D_OUT), F32),
    )(e_feats, ga, gb, W1e[:E_IN], W2e, b2e.reshape(1, LAT), W3e,
      b3e.reshape(1, D_OUT))

    dst2d = dst.reshape(NCHUNK, CH)
    z2d = jnp.zeros((125, D_OUT), F32)
    z1d = jnp.zeros((2000,), F32)
    ones = jnp.ones((CH,), F32)

    psum, pdeg = pl.kernel(
        functools.partial(_scatter_body, nc, nw, nfull, rem),
        out_type=[
            jax.ShapeDtypeStruct((2, N, D_OUT), F32),
            jax.ShapeDtypeStruct((2, N), F32),
        ],
        mesh=mesh,
        scratch_types=[
            pltpu.VMEM_SHARED((N, D_OUT), F32),
            pltpu.VMEM_SHARED((N,), F32),
            pltpu.VMEM((1, CH), jnp.int32),
            pltpu.VMEM((CH, D_OUT), F32),
            pltpu.VMEM((CH,), F32),
        ],
    )(dst2d, e_out, z2d, z1d, ones)

    n_out = pl.pallas_call(
        _nmlp_body,
        grid=(N // KN,),
        in_specs=[
            pl.BlockSpec((KN, D_IN), lambda i: (i, 0)),
            pl.BlockSpec((2, KN, D_OUT), lambda i: (0, i, 0)),
            pl.BlockSpec((2, N), lambda i: (0, 0)),
            pl.BlockSpec((D_IN, LAT), lambda i: (0, 0)),
            pl.BlockSpec((D_OUT, LAT), lambda i: (0, 0)),
            pl.BlockSpec((1, LAT), lambda i: (0, 0)),
            pl.BlockSpec((LAT, LAT), lambda i: (0, 0)),
            pl.BlockSpec((1, LAT), lambda i: (0, 0)),
            pl.BlockSpec((LAT, D_OUT), lambda i: (0, 0)),
            pl.BlockSpec((1, D_OUT), lambda i: (0, 0)),
        ],
        out_specs=pl.BlockSpec((KN, D_OUT), lambda i: (i, 0)),
        out_shape=jax.ShapeDtypeStruct((N, D_OUT), F32),
    )(n_feats, psum, pdeg, W1n[:D_IN], W1n[D_IN:], b1n.reshape(1, LAT),
      W2n, b2n.reshape(1, LAT), W3n, b3n.reshape(1, D_OUT))

    return (n_out, e_out)


# speedup vs baseline: 3.9410x; 3.9410x over previous
import functools

import jax
import jax.numpy as jnp
from jax import lax
from jax.experimental import pallas as pl
from jax.experimental.pallas import tpu as pltpu
from jax.experimental.pallas import tpu_sc as plsc

F32 = jnp.float32

N = 10000
E = 320000
E_IN = 16
D_IN = 128
LAT = 16
D_OUT = 128

CH = 128
NCHUNK = E // CH
NP = 10240
KN = 2048
KE = 2000


def _dot(a, b):
    return lax.dot_general(a, b, (((1,), (0,)), ((), ())),
                           preferred_element_type=F32)


def _prep_body(nf_ref, wcat_ref, bcat_ref, t_ref):
    t_ref[...] = _dot(nf_ref[...], wcat_ref[...]) + bcat_ref[...]


def _emlp_body(ef_ref, g_ref, w1_ref, w2_ref, b2_ref, w3_ref,
               b3_ref, out_ref):
    h = jnp.maximum(_dot(ef_ref[...], w1_ref[...]) + g_ref[...], 0.0)
    h = jnp.maximum(_dot(h, w2_ref[...]) + b2_ref[...], 0.0)
    z = _dot(h, w3_ref[...]) + b3_ref[...]
    out_ref[...] = 1.0 / (1.0 + jnp.exp(-z))


def _nmlp_body(nf_ref, ps0_ref, ps1_ref, pd0_ref, pd1_ref, w1t_ref, w1b_ref,
               b1_ref, w2_ref, b2_ref, w3_ref, b3_ref, out_ref):
    s = ps0_ref[...] + ps1_ref[...]
    d = pd0_ref[...] + pd1_ref[...]
    recip = lax.broadcast_in_dim(1.0 / jnp.maximum(d, 1.0), (KN, D_OUT), (0,))
    h_n = s * recip
    h = jnp.maximum(_dot(nf_ref[...], w1t_ref[...]) + _dot(h_n, w1b_ref[...])
                    + b1_ref[...], 0.0)
    h = jnp.maximum(_dot(h, w2_ref[...]) + b2_ref[...], 0.0)
    z = _dot(h, w3_ref[...]) + b3_ref[...]
    out_ref[...] = 1.0 / (1.0 + jnp.exp(-z))


def _gather_body(nc, nw, nfull, rem,
                 src_hbm, dst_hbm, t_hbm, g_hbm,
                 sv, dv, bs, bd, bufg, sem_a, sem_b):
    sid = lax.axis_index("s")
    wid = sid * nc + lax.axis_index("c")
    n_my = nfull + jnp.where(wid < rem, 1, 0)

    def chunk(j, carry):
        off = CH * (wid + nw * j)
        pltpu.sync_copy(src_hbm.at[pl.ds(off, CH)], sv)
        pltpu.sync_copy(dst_hbm.at[pl.ds(off, CH)], dv)
        ca = pltpu.async_copy(t_hbm.at[sv], bs, sem_a)
        cb = pltpu.async_copy(t_hbm.at[dv], bd, sem_b)
        ca.wait()
        cb.wait()

        def row(i, c):
            bufg[i, :] = bs[i, pl.ds(0, LAT)] + bd[i, pl.ds(LAT, LAT)]
            return c

        lax.fori_loop(0, CH, row, 0)
        pltpu.sync_copy(bufg, g_hbm.at[pl.ds(off, CH)])
        return carry

    lax.fori_loop(0, n_my, chunk, 0)


def _scatter_body(nc, nw, nfull, rem,
                  dst2d_hbm, eo_hbm, z2d_hbm, z1d_hbm, ones_hbm,
                  psum0_hbm, psum1_hbm, pdeg0_hbm, pdeg1_hbm,
                  sums_sp, deg_sp, idxv, rows, onesv, dzv):
    cid = lax.axis_index("c")
    sid = lax.axis_index("s")
    wid = sid * nc + cid
    n_my = nfull + jnp.where(wid < rem, 1, 0)

    for k in range(5):
        pltpu.sync_copy(z2d_hbm, sums_sp.at[pl.ds(640 * sid + 128 * k, 128)])

    pltpu.sync_copy(z1d_hbm, dzv)
    pltpu.sync_copy(dzv.at[pl.ds(0, 640)], deg_sp.at[pl.ds(640 * sid, 640)])

    pltpu.sync_copy(ones_hbm, onesv)
    plsc.subcore_barrier()

    def chunk(j, carry):
        c = wid + nw * j
        pltpu.sync_copy(dst2d_hbm.at[c], idxv.at[0])
        pltpu.sync_copy(eo_hbm.at[pl.ds(CH * c, CH)], rows)
        pltpu.sync_copy(rows, sums_sp.at[idxv.at[0]], add=True)
        pltpu.sync_copy(onesv, deg_sp.at[idxv.at[0]], add=True)
        return carry

    lax.fori_loop(0, n_my, chunk, 0)
    plsc.subcore_barrier()

    @pl.when(sid < 10)
    def _():
        pltpu.sync_copy(deg_sp.at[pl.ds(1024 * sid, 1024)], dzv)

        @pl.when(cid == 0)
        def _():
            pltpu.sync_copy(sums_sp.at[pl.ds(1024 * sid, 1024)],
                            psum0_hbm.at[pl.ds(1024 * sid, 1024)])
            pltpu.sync_copy(dzv, pdeg0_hbm.at[pl.ds(1024 * sid, 1024)])

        @pl.when(cid == 1)
        def _():
            pltpu.sync_copy(sums_sp.at[pl.ds(1024 * sid, 1024)],
                            psum1_hbm.at[pl.ds(1024 * sid, 1024)])
            pltpu.sync_copy(dzv, pdeg1_hbm.at[pl.ds(1024 * sid, 1024)])


def kernel(n_feats, e_feats, edge_index, W1e, b1e, W2e, b2e, W3e, b3e,
           W1n, b1n, W2n, b2n, W3n, b3n):
    src = edge_index[0]
    dst = edge_index[1]

    wcat = jnp.concatenate(
        [W1e[E_IN:E_IN + D_IN], W1e[E_IN + D_IN:],
         jnp.zeros((D_IN, D_IN - 2 * LAT), F32)], axis=1)
    bcat = jnp.concatenate(
        [jnp.zeros((LAT,), F32), b1e,
         jnp.zeros((D_IN - 2 * LAT,), F32)]).reshape(1, D_IN)

    t_tab = pl.pallas_call(
        _prep_body,
        grid=(1,),
        in_specs=[
            pl.BlockSpec((N, D_IN), lambda i: (0, 0)),
            pl.BlockSpec((D_IN, D_IN), lambda i: (0, 0)),
            pl.BlockSpec((1, D_IN), lambda i: (0, 0)),
        ],
        out_specs=pl.BlockSpec((N, D_IN), lambda i: (0, 0)),
        out_shape=jax.ShapeDtypeStruct((N, D_IN), F32),
    )(n_feats, wcat, bcat)

    info = plsc.get_sparse_core_info()
    nc, ns = info.num_cores, info.num_subcores
    nw = nc * ns
    nfull, rem = NCHUNK // nw, NCHUNK % nw
    mesh = plsc.VectorSubcoreMesh(core_axis_name="c", subcore_axis_name="s")

    g = pl.kernel(
        functools.partial(_gather_body, nc, nw, nfull, rem),
        out_type=jax.ShapeDtypeStruct((E, LAT), F32),
        mesh=mesh,
        scratch_types=[
            pltpu.VMEM((CH,), jnp.int32),
            pltpu.VMEM((CH,), jnp.int32),
            pltpu.VMEM((CH, D_IN), F32),
            pltpu.VMEM((CH, D_IN), F32),
            pltpu.VMEM((CH, LAT), F32),
            pltpu.SemaphoreType.DMA,
            pltpu.SemaphoreType.DMA,
        ],
    )(src, dst, t_tab)

    e_out = pl.pallas_call(
        _emlp_body,
        grid=(E // KE,),
        in_specs=[
            pl.BlockSpec((KE, E_IN), lambda i: (i, 0)),
            pl.BlockSpec((KE, LAT), lambda i: (i, 0)),
            pl.BlockSpec((E_IN, LAT), lambda i: (0, 0)),
            pl.BlockSpec((LAT, LAT), lambda i: (0, 0)),
            pl.BlockSpec((1, LAT), lambda i: (0, 0)),
            pl.BlockSpec((LAT, D_OUT), lambda i: (0, 0)),
            pl.BlockSpec((1, D_OUT), lambda i: (0, 0)),
        ],
        out_specs=pl.BlockSpec((KE, D_OUT), lambda i: (i, 0)),
        out_shape=jax.ShapeDtypeStruct((E, D_OUT), F32),
    )(e_feats, g, W1e[:E_IN], W2e, b2e.reshape(1, LAT), W3e,
      b3e.reshape(1, D_OUT))

    dst2d = dst.reshape(NCHUNK, CH)
    z2d = jnp.zeros((CH, D_OUT), F32)
    z1d = jnp.zeros((1024,), F32)
    ones = jnp.ones((CH,), F32)

    psum0, psum1, pdeg0, pdeg1 = pl.kernel(
        functools.partial(_scatter_body, nc, nw, nfull, rem),
        out_type=[
            jax.ShapeDtypeStruct((NP, D_OUT), F32),
            jax.ShapeDtypeStruct((NP, D_OUT), F32),
            jax.ShapeDtypeStruct((NP,), F32),
            jax.ShapeDtypeStruct((NP,), F32),
        ],
        mesh=mesh,
        scratch_types=[
            pltpu.VMEM_SHARED((NP, D_OUT), F32),
            pltpu.VMEM_SHARED((NP,), F32),
            pltpu.VMEM((1, CH), jnp.int32),
            pltpu.VMEM((CH, D_OUT), F32),
            pltpu.VMEM((CH,), F32),
            pltpu.VMEM((1024,), F32),
        ],
    )(dst2d, e_out, z2d, z1d, ones)

    n_out = pl.pallas_call(
        _nmlp_body,
        grid=(NP // KN,),
        in_specs=[
            pl.BlockSpec((KN, D_IN), lambda i: (i, 0)),
            pl.BlockSpec((KN, D_OUT), lambda i: (i, 0)),
            pl.BlockSpec((KN, D_OUT), lambda i: (i, 0)),
            pl.BlockSpec((KN,), lambda i: (i,)),
            pl.BlockSpec((KN,), lambda i: (i,)),
            pl.BlockSpec((D_IN, LAT), lambda i: (0, 0)),
            pl.BlockSpec((D_OUT, LAT), lambda i: (0, 0)),
            pl.BlockSpec((1, LAT), lambda i: (0, 0)),
            pl.BlockSpec((LAT, LAT), lambda i: (0, 0)),
            pl.BlockSpec((1, LAT), lambda i: (0, 0)),
            pl.BlockSpec((LAT, D_OUT), lambda i: (0, 0)),
            pl.BlockSpec((1, D_OUT), lambda i: (0, 0)),
        ],
        out_specs=pl.BlockSpec((KN, D_OUT), lambda i: (i, 0)),
        out_shape=jax.ShapeDtypeStruct((N, D_OUT), F32),
    )(n_feats, psum0, psum1, pdeg0, pdeg1, W1n[:D_IN], W1n[D_IN:],
      b1n.reshape(1, LAT), W2n, b2n.reshape(1, LAT), W3n,
      b3n.reshape(1, D_OUT))

    return (n_out, e_out)


# speedup vs baseline: 5.2949x; 1.3436x over previous
import functools

import jax
import jax.numpy as jnp
from jax import lax
from jax.experimental import pallas as pl
from jax.experimental.pallas import tpu as pltpu
from jax.experimental.pallas import tpu_sc as plsc

F32 = jnp.float32

N = 10000
E = 320000
E_IN = 16
D_IN = 128
LAT = 16
D_OUT = 128

CH = 128
NCHUNK = E // CH
NP = 10240
KN = 2048
KE = 2000


def _dot(a, b):
    return lax.dot_general(a, b, (((1,), (0,)), ((), ())),
                           preferred_element_type=F32)


def _prep_body(nf_ref, wcat_ref, bcat_ref, t_ref):
    t_ref[...] = _dot(nf_ref[...], wcat_ref[...]) + bcat_ref[...]


def _emlp_body(ef_ref, g_ref, w1_ref, w2_ref, b2_ref, w3_ref,
               b3_ref, out_ref):
    h = jnp.maximum(_dot(ef_ref[...], w1_ref[...]) + g_ref[...], 0.0)
    h = jnp.maximum(_dot(h, w2_ref[...]) + b2_ref[...], 0.0)
    z = _dot(h, w3_ref[...]) + b3_ref[...]
    out_ref[...] = 1.0 / (1.0 + jnp.exp(-z))


def _nmlp_body(nf_ref, ps0_ref, ps1_ref, pd0_ref, pd1_ref, w1t_ref, w1b_ref,
               b1_ref, w2_ref, b2_ref, w3_ref, b3_ref, out_ref):
    s = ps0_ref[...] + ps1_ref[...]
    d = pd0_ref[...] + pd1_ref[...]
    recip = lax.broadcast_in_dim(1.0 / jnp.maximum(d, 1.0), (KN, D_OUT), (0,))
    h_n = s * recip
    h = jnp.maximum(_dot(nf_ref[...], w1t_ref[...]) + _dot(h_n, w1b_ref[...])
                    + b1_ref[...], 0.0)
    h = jnp.maximum(_dot(h, w2_ref[...]) + b2_ref[...], 0.0)
    z = _dot(h, w3_ref[...]) + b3_ref[...]
    out_ref[...] = 1.0 / (1.0 + jnp.exp(-z))


def _gather_body(nc, nw, nfull, rem,
                 src_hbm, dst_hbm, t_hbm, g_hbm,
                 sv_all, dv_all, bs0, bd0, bs1, bd1, g0, g1,
                 sem_g0, sem_g1, sem_w0, sem_w1):
    assert nfull % 2 == 0
    sid = lax.axis_index("s")
    wid = sid * nc + lax.axis_index("c")
    extra = wid < rem
    c0 = wid * nfull + jnp.minimum(wid, rem)

    pltpu.sync_copy(src_hbm.at[pl.ds(c0 * CH, nfull * CH)],
                    sv_all.at[pl.ds(0, nfull * CH)])
    pltpu.sync_copy(dst_hbm.at[pl.ds(c0 * CH, nfull * CH)],
                    dv_all.at[pl.ds(0, nfull * CH)])

    @pl.when(extra)
    def _():
        pltpu.sync_copy(src_hbm.at[pl.ds((c0 + nfull) * CH, CH)],
                        sv_all.at[pl.ds(nfull * CH, CH)])
        pltpu.sync_copy(dst_hbm.at[pl.ds((c0 + nfull) * CH, CH)],
                        dv_all.at[pl.ds(nfull * CH, CH)])

    bufs = ((bs0, bd0, g0, sem_g0, sem_w0), (bs1, bd1, g1, sem_g1, sem_w1))

    def issue(jl, b):
        bs, bd, _, sem_g, _ = bufs[b]
        pltpu.async_copy(t_hbm.at[sv_all.at[pl.ds(CH * jl, CH)]], bs, sem_g)
        pltpu.async_copy(t_hbm.at[dv_all.at[pl.ds(CH * jl, CH)]], bd, sem_g)

    def wait_g(b):
        bs, bd, _, sem_g, _ = bufs[b]
        pltpu.make_async_copy(t_hbm.at[pl.ds(0, CH)], bs, sem_g).wait()
        pltpu.make_async_copy(t_hbm.at[pl.ds(0, CH)], bd, sem_g).wait()

    def extract(b):
        bs, bd, g, _, _ = bufs[b]

        def row(i0, c):
            for u in range(8):
                i = 8 * i0 + u
                g[i, :] = bs[i, pl.ds(0, LAT)] + bd[i, pl.ds(LAT, LAT)]
            return c

        lax.fori_loop(0, CH // 8, row, 0)

    def writeback(jl, b, p):
        _, _, g, _, sem_w = bufs[b]

        @pl.when(p > 0)
        def _():
            pltpu.make_async_copy(g_hbm.at[pl.ds(0, CH)], g, sem_w).wait()

        pltpu.async_copy(g, g_hbm.at[pl.ds(CH * (c0 + jl), CH)], sem_w)

    issue(0, 0)

    def pair(p, carry):
        j0 = 2 * p
        issue(j0 + 1, 1)
        wait_g(0)
        extract(0)
        writeback(j0, 0, p)
        issue(jnp.minimum(j0 + 2, nfull - 1), 0)
        wait_g(1)
        extract(1)
        writeback(j0 + 1, 1, p)
        return carry

    lax.fori_loop(0, nfull // 2, pair, 0)

    wait_g(0)
    pltpu.make_async_copy(g_hbm.at[pl.ds(0, CH)], g0, sem_w0).wait()
    pltpu.make_async_copy(g_hbm.at[pl.ds(0, CH)], g1, sem_w1).wait()

    @pl.when(extra)
    def _():
        issue(nfull, 0)
        wait_g(0)
        extract(0)
        pltpu.sync_copy(g0, g_hbm.at[pl.ds(CH * (c0 + nfull), CH)])


def _scatter_body(nc, nw, nfull, rem,
                  dst2d_hbm, eo_hbm, z2d_hbm, z1d_hbm, ones_hbm,
                  psum0_hbm, psum1_hbm, pdeg0_hbm, pdeg1_hbm,
                  sums_sp, deg_sp, iv, rb0, rb1, onesv, dzv,
                  sem_r0, sem_r1):
    assert nfull % 2 == 0
    cid = lax.axis_index("c")
    sid = lax.axis_index("s")
    wid = sid * nc + cid
    extra = wid < rem
    c0 = wid * nfull + jnp.minimum(wid, rem)

    for k in range(5):
        pltpu.sync_copy(z2d_hbm, sums_sp.at[pl.ds(640 * sid + 128 * k, 128)])

    pltpu.sync_copy(z1d_hbm, dzv)
    pltpu.sync_copy(dzv.at[pl.ds(0, 640)], deg_sp.at[pl.ds(640 * sid, 640)])

    pltpu.sync_copy(ones_hbm, onesv)
    plsc.subcore_barrier()

    bufs = ((rb0, sem_r0), (rb1, sem_r1))

    def issue(jl, b):
        rb, sem_r = bufs[b]
        pltpu.async_copy(dst2d_hbm.at[c0 + jl], iv.at[b], sem_r)
        pltpu.async_copy(eo_hbm.at[pl.ds(CH * (c0 + jl), CH)], rb, sem_r)

    def wait_r(b):
        rb, sem_r = bufs[b]
        pltpu.make_async_copy(dst2d_hbm.at[0], iv.at[b], sem_r).wait()
        pltpu.make_async_copy(eo_hbm.at[pl.ds(0, CH)], rb, sem_r).wait()

    def scat(b):
        rb, _ = bufs[b]
        pltpu.sync_copy(rb, sums_sp.at[iv.at[b]], add=True)
        pltpu.sync_copy(onesv, deg_sp.at[iv.at[b]], add=True)

    issue(0, 0)

    def pair(p, carry):
        j0 = 2 * p
        issue(j0 + 1, 1)
        wait_r(0)
        scat(0)
        issue(jnp.minimum(j0 + 2, nfull - 1), 0)
        wait_r(1)
        scat(1)
        return carry

    lax.fori_loop(0, nfull // 2, pair, 0)
    wait_r(0)

    @pl.when(extra)
    def _():
        issue(nfull, 0)
        wait_r(0)
        scat(0)

    plsc.subcore_barrier()

    @pl.when(sid < 10)
    def _():
        pltpu.sync_copy(deg_sp.at[pl.ds(1024 * sid, 1024)], dzv)

        @pl.when(cid == 0)
        def _():
            pltpu.sync_copy(sums_sp.at[pl.ds(1024 * sid, 1024)],
                            psum0_hbm.at[pl.ds(1024 * sid, 1024)])
            pltpu.sync_copy(dzv, pdeg0_hbm.at[pl.ds(1024 * sid, 1024)])

        @pl.when(cid == 1)
        def _():
            pltpu.sync_copy(sums_sp.at[pl.ds(1024 * sid, 1024)],
                            psum1_hbm.at[pl.ds(1024 * sid, 1024)])
            pltpu.sync_copy(dzv, pdeg1_hbm.at[pl.ds(1024 * sid, 1024)])


def kernel(n_feats, e_feats, edge_index, W1e, b1e, W2e, b2e, W3e, b3e,
           W1n, b1n, W2n, b2n, W3n, b3n):
    src = edge_index[0]
    dst = edge_index[1]

    wcat = jnp.concatenate(
        [W1e[E_IN:E_IN + D_IN], W1e[E_IN + D_IN:],
         jnp.zeros((D_IN, D_IN - 2 * LAT), F32)], axis=1)
    bcat = jnp.concatenate(
        [jnp.zeros((LAT,), F32), b1e,
         jnp.zeros((D_IN - 2 * LAT,), F32)]).reshape(1, D_IN)

    t_tab = pl.pallas_call(
        _prep_body,
        grid=(1,),
        in_specs=[
            pl.BlockSpec((N, D_IN), lambda i: (0, 0)),
            pl.BlockSpec((D_IN, D_IN), lambda i: (0, 0)),
            pl.BlockSpec((1, D_IN), lambda i: (0, 0)),
        ],
        out_specs=pl.BlockSpec((N, D_IN), lambda i: (0, 0)),
        out_shape=jax.ShapeDtypeStruct((N, D_IN), F32),
    )(n_feats, wcat, bcat)

    info = plsc.get_sparse_core_info()
    nc, ns = info.num_cores, info.num_subcores
    nw = nc * ns
    nfull, rem = NCHUNK // nw, NCHUNK % nw
    mesh = plsc.VectorSubcoreMesh(core_axis_name="c", subcore_axis_name="s")

    g = pl.kernel(
        functools.partial(_gather_body, nc, nw, nfull, rem),
        out_type=jax.ShapeDtypeStruct((E, LAT), F32),
        mesh=mesh,
        scratch_types=[
            pltpu.VMEM(((nfull + 1) * CH,), jnp.int32),
            pltpu.VMEM(((nfull + 1) * CH,), jnp.int32),
            pltpu.VMEM((CH, D_IN), F32),
            pltpu.VMEM((CH, D_IN), F32),
            pltpu.VMEM((CH, D_IN), F32),
            pltpu.VMEM((CH, D_IN), F32),
            pltpu.VMEM((CH, LAT), F32),
            pltpu.VMEM((CH, LAT), F32),
            pltpu.SemaphoreType.DMA,
            pltpu.SemaphoreType.DMA,
            pltpu.SemaphoreType.DMA,
            pltpu.SemaphoreType.DMA,
        ],
    )(src, dst, t_tab)

    e_out = pl.pallas_call(
        _emlp_body,
        grid=(E // KE,),
        in_specs=[
            pl.BlockSpec((KE, E_IN), lambda i: (i, 0)),
            pl.BlockSpec((KE, LAT), lambda i: (i, 0)),
            pl.BlockSpec((E_IN, LAT), lambda i: (0, 0)),
            pl.BlockSpec((LAT, LAT), lambda i: (0, 0)),
            pl.BlockSpec((1, LAT), lambda i: (0, 0)),
            pl.BlockSpec((LAT, D_OUT), lambda i: (0, 0)),
            pl.BlockSpec((1, D_OUT), lambda i: (0, 0)),
        ],
        out_specs=pl.BlockSpec((KE, D_OUT), lambda i: (i, 0)),
        out_shape=jax.ShapeDtypeStruct((E, D_OUT), F32),
    )(e_feats, g, W1e[:E_IN], W2e, b2e.reshape(1, LAT), W3e,
      b3e.reshape(1, D_OUT))

    dst2d = dst.reshape(NCHUNK, CH)
    z2d = jnp.zeros((CH, D_OUT), F32)
    z1d = jnp.zeros((1024,), F32)
    ones = jnp.ones((CH,), F32)

    psum0, psum1, pdeg0, pdeg1 = pl.kernel(
        functools.partial(_scatter_body, nc, nw, nfull, rem),
        out_type=[
            jax.ShapeDtypeStruct((NP, D_OUT), F32),
            jax.ShapeDtypeStruct((NP, D_OUT), F32),
            jax.ShapeDtypeStruct((NP,), F32),
            jax.ShapeDtypeStruct((NP,), F32),
        ],
        mesh=mesh,
        scratch_types=[
            pltpu.VMEM_SHARED((NP, D_OUT), F32),
            pltpu.VMEM_SHARED((NP,), F32),
            pltpu.VMEM((2, CH), jnp.int32),
            pltpu.VMEM((CH, D_OUT), F32),
            pltpu.VMEM((CH, D_OUT), F32),
            pltpu.VMEM((CH,), F32),
            pltpu.VMEM((1024,), F32),
            pltpu.SemaphoreType.DMA,
            pltpu.SemaphoreType.DMA,
        ],
    )(dst2d, e_out, z2d, z1d, ones)

    n_out = pl.pallas_call(
        _nmlp_body,
        grid=(NP // KN,),
        in_specs=[
            pl.BlockSpec((KN, D_IN), lambda i: (i, 0)),
            pl.BlockSpec((KN, D_OUT), lambda i: (i, 0)),
            pl.BlockSpec((KN, D_OUT), lambda i: (i, 0)),
            pl.BlockSpec((KN,), lambda i: (i,)),
            pl.BlockSpec((KN,), lambda i: (i,)),
            pl.BlockSpec((D_IN, LAT), lambda i: (0, 0)),
            pl.BlockSpec((D_OUT, LAT), lambda i: (0, 0)),
            pl.BlockSpec((1, LAT), lambda i: (0, 0)),
            pl.BlockSpec((LAT, LAT), lambda i: (0, 0)),
            pl.BlockSpec((1, LAT), lambda i: (0, 0)),
            pl.BlockSpec((LAT, D_OUT), lambda i: (0, 0)),
            pl.BlockSpec((1, D_OUT), lambda i: (0, 0)),
        ],
        out_specs=pl.BlockSpec((KN, D_OUT), lambda i: (i, 0)),
        out_shape=jax.ShapeDtypeStruct((N, D_OUT), F32),
    )(n_feats, psum0, psum1, pdeg0, pdeg1, W1n[:D_IN], W1n[D_IN:],
      b1n.reshape(1, LAT), W2n, b2n.reshape(1, LAT), W3n,
      b3n.reshape(1, D_OUT))

    return (n_out, e_out)
